# Initial kernel scaffold; baseline (speedup 1.0000x reference)
import functools

import jax
import jax.numpy as jnp
from jax import lax
from jax.experimental import pallas as pl
from jax.experimental.pallas import tpu as pltpu
from jax.experimental.pallas import tpu_sc as plsc

N_NODES = 10000
DIM_IN = 128
DIM_H = 64
DIM_O = 6
N_GRAPH = 64
N_EDGE = 320000

NC, NS, LANES = 2, 16, 16
NW = NC * NS
NP = 10240
ROWS_W = NP // NS
CH = 128
NCH_W = 82
EP = NW * NCH_W * CH
BLK = 640
GRID = NP // BLK

_mesh = plsc.VectorSubcoreMesh(
    core_axis_name="c", subcore_axis_name="s", num_cores=NC, num_subcores=NS)


@functools.partial(
    pl.kernel,
    out_type=jax.ShapeDtypeStruct((NC * NP, LANES), jnp.float32),
    mesh=_mesh,
    scratch_types=[
        pltpu.VMEM((CH, LANES), jnp.float32),
        pltpu.VMEM((CH, LANES), jnp.float32),
        pltpu.VMEM((CH,), jnp.int32),
        pltpu.VMEM_SHARED((NP, LANES), jnp.float32),
    ],
)
def _deg_kernel(dst_hbm, out_hbm, zb, ones_v, didx, cnt_sp):
    c = lax.axis_index("c")
    s = lax.axis_index("s")
    w = c * NS + s

    def fill(i, _):
        zb[i, :] = jnp.zeros((LANES,), jnp.float32)
        ones_v[i, :] = jnp.ones((LANES,), jnp.float32)
        return 0

    lax.fori_loop(0, CH, fill, 0)
    for k in range(ROWS_W // CH):
        pltpu.sync_copy(zb, cnt_sp.at[pl.ds(s * ROWS_W + k * CH, CH)])
    plsc.subcore_barrier()

    def chunk(j, _):
        base = pl.multiple_of((w * NCH_W + j) * CH, CH)
        pltpu.sync_copy(dst_hbm.at[pl.ds(base, CH)], didx)
        pltpu.sync_copy(ones_v, cnt_sp.at[didx], add=True)
        return 0

    lax.fori_loop(0, NCH_W, chunk, 0)
    plsc.subcore_barrier()
    pltpu.sync_copy(cnt_sp.at[pl.ds(s * ROWS_W, ROWS_W)],
                    out_hbm.at[pl.ds(c * NP + s * ROWS_W, ROWS_W)])


@functools.partial(
    pl.kernel,
    out_type=jax.ShapeDtypeStruct((NC * NP, DIM_H), jnp.float32),
    mesh=_mesh,
    scratch_types=[
        pltpu.VMEM((CH, DIM_H), jnp.float32),
        pltpu.VMEM((2, CH, DIM_H), jnp.float32),
        pltpu.VMEM((2, CH), jnp.int32),
        pltpu.VMEM((2, CH), jnp.int32),
        pltpu.SemaphoreType.DMA,
        pltpu.VMEM_SHARED((NP, DIM_H), jnp.float32),
    ],
)
def _agg_kernel(g_hbm, src_hbm, dst_hbm, out_hbm, zb, rows, sidx, didx, sem,
                acc_sp):
    c = lax.axis_index("c")
    s = lax.axis_index("s")
    w = c * NS + s

    def fill(i, _):
        for k in range(DIM_H // LANES):
            zb[i, pl.ds(k * LANES, LANES)] = jnp.zeros((LANES,), jnp.float32)
        return 0

    lax.fori_loop(0, CH, fill, 0)
    for k in range(ROWS_W // CH):
        pltpu.sync_copy(zb, acc_sp.at[pl.ds(s * ROWS_W + k * CH, CH)])
    plsc.subcore_barrier()

    base0 = w * NCH_W * CH

    def chunk(j, _):
        base = pl.multiple_of(base0 + j * CH, CH)
        pltpu.sync_copy(src_hbm.at[pl.ds(base, CH)], sidx.at[0])
        pltpu.sync_copy(dst_hbm.at[pl.ds(base, CH)], didx.at[0])
        pltpu.async_copy(g_hbm.at[sidx.at[0]], rows.at[0], sem).wait()
        pltpu.sync_copy(rows.at[0], acc_sp.at[didx.at[0]], add=True)
        return 0

    lax.fori_loop(0, NCH_W, chunk, 0)
    plsc.subcore_barrier()
    pltpu.sync_copy(acc_sp.at[pl.ds(s * ROWS_W, ROWS_W)],
                    out_hbm.at[pl.ds(c * NP + s * ROWS_W, ROWS_W)])


def _dinv_block(d0_ref, d1_ref):
    i = pl.program_id(0)
    deg = d0_ref[:, 0:1] + d1_ref[:, 0:1]
    rows = i * BLK + lax.broadcasted_iota(jnp.int32, (BLK, 1), 0)
    ok = (rows < N_NODES) & (deg > 0.0)
    return jnp.where(ok, lax.rsqrt(jnp.maximum(deg, 1e-30)), 0.0)


def _tc1_body(x_ref, w1_ref, d0_ref, d1_ref, g_ref):
    dinv = _dinv_block(d0_ref, d1_ref)
    g = jnp.dot(x_ref[...], w1_ref[...], preferred_element_type=jnp.float32)
    g_ref[...] = g * dinv


def _tc1(xp, W1, d0, d1):
    return pl.pallas_call(
        _tc1_body,
        grid=(GRID,),
        in_specs=[
            pl.BlockSpec((BLK, DIM_IN), lambda i: (i, 0)),
            pl.BlockSpec((DIM_IN, DIM_H), lambda i: (0, 0)),
            pl.BlockSpec((BLK, LANES), lambda i: (i, 0)),
            pl.BlockSpec((BLK, LANES), lambda i: (i, 0)),
        ],
        out_specs=pl.BlockSpec((BLK, DIM_H), lambda i: (i, 0)),
        out_shape=jax.ShapeDtypeStruct((NP, DIM_H), jnp.float32),
    )(xp, W1, d0, d1)


def _tc2_body(p0_ref, p1_ref, d0_ref, d1_ref, b1_ref, w2_ref, g_ref):
    dinv = _dinv_block(d0_ref, d1_ref)
    a = (p0_ref[...] + p1_ref[...]) * dinv + b1_ref[...]
    h = jnp.maximum(a, 0.0)
    g_ref[...] = jnp.dot(h, w2_ref[...],
                         preferred_element_type=jnp.float32) * dinv


def _tc2(p0, p1, d0, d1, b1r, W2):
    return pl.pallas_call(
        _tc2_body,
        grid=(GRID,),
        in_specs=[
            pl.BlockSpec((BLK, DIM_H), lambda i: (i, 0)),
            pl.BlockSpec((BLK, DIM_H), lambda i: (i, 0)),
            pl.BlockSpec((BLK, LANES), lambda i: (i, 0)),
            pl.BlockSpec((BLK, LANES), lambda i: (i, 0)),
            pl.BlockSpec((1, DIM_H), lambda i: (0, 0)),
            pl.BlockSpec((DIM_H, DIM_H), lambda i: (0, 0)),
        ],
        out_specs=pl.BlockSpec((BLK, DIM_H), lambda i: (i, 0)),
        out_shape=jax.ShapeDtypeStruct((NP, DIM_H), jnp.float32),
    )(p0, p1, d0, d1, b1r, W2)


def _tc3_body(p0_ref, p1_ref, d0_ref, d1_ref, b2_ref, bt_ref, wl_ref, bl_ref,
              fin_ref, acc):
    i = pl.program_id(0)
    dinv = _dinv_block(d0_ref, d1_ref)
    h2 = jnp.maximum((p0_ref[...] + p1_ref[...]) * dinv + b2_ref[...], 0.0)
    bt = bt_ref[0]
    gids = lax.broadcasted_iota(jnp.int32, (N_GRAPH, BLK), 0)
    oh = (bt == gids).astype(jnp.float32)
    haug = jnp.concatenate([h2, jnp.ones((BLK, DIM_H), jnp.float32)], axis=1)
    part = jnp.dot(oh, haug, preferred_element_type=jnp.float32)

    @pl.when(i == 0)
    def _():
        acc[...] = part

    @pl.when(i > 0)
    def _():
        acc[...] += part

    @pl.when(i == GRID - 1)
    def _():
        sums = acc[:, :DIM_H]
        cnt = acc[:, DIM_H:DIM_H + 1]
        pooled = sums / jnp.maximum(cnt, 1.0)
        fin_ref[...] = jnp.dot(pooled, wl_ref[...],
                               preferred_element_type=jnp.float32) + bl_ref[...]


def _tc3(p0, p1, d0, d1, b2r, batchp, wlp, blp):
    return pl.pallas_call(
        _tc3_body,
        grid=(GRID,),
        in_specs=[
            pl.BlockSpec((BLK, DIM_H), lambda i: (i, 0)),
            pl.BlockSpec((BLK, DIM_H), lambda i: (i, 0)),
            pl.BlockSpec((BLK, LANES), lambda i: (i, 0)),
            pl.BlockSpec((BLK, LANES), lambda i: (i, 0)),
            pl.BlockSpec((1, DIM_H), lambda i: (0, 0)),
            pl.BlockSpec((1, 1, BLK), lambda i: (i, 0, 0)),
            pl.BlockSpec((DIM_H, 128), lambda i: (0, 0)),
            pl.BlockSpec((1, 128), lambda i: (0, 0)),
        ],
        out_specs=pl.BlockSpec((N_GRAPH, 128), lambda i: (0, 0)),
        out_shape=jax.ShapeDtypeStruct((N_GRAPH, 128), jnp.float32),
        scratch_shapes=[pltpu.VMEM((N_GRAPH, 128), jnp.float32)],
    )(p0, p1, d0, d1, b2r, batchp, wlp, blp)


def kernel(x, edge_index, batch, W1, b1, W2, b2, Wlin, blin):
    loop = jnp.arange(N_NODES, dtype=jnp.int32)
    npad = EP - (N_EDGE + N_NODES)
    pad_rows = N_NODES + (jnp.arange(npad, dtype=jnp.int32) % (NP - N_NODES))
    src = jnp.concatenate([edge_index[0], loop, pad_rows])
    dst = jnp.concatenate([edge_index[1], loop, pad_rows])

    deg2 = _deg_kernel(dst)
    d0, d1 = deg2[:NP], deg2[NP:]

    xp = jnp.pad(x, ((0, NP - N_NODES), (0, 0)))
    g1 = _tc1(xp, W1, d0, d1)
    a1 = _agg_kernel(g1, src, dst)
    g2 = _tc2(a1[:NP], a1[NP:], d0, d1, b1.reshape(1, DIM_H), W2)
    a2 = _agg_kernel(g2, src, dst)

    batchp = jnp.pad(batch, (0, NP - N_NODES),
                     constant_values=N_GRAPH).reshape(GRID, 1, BLK)
    wlp = jnp.pad(Wlin, ((0, 0), (0, 128 - DIM_O)))
    blp = jnp.pad(blin, (0, 128 - DIM_O)).reshape(1, 128)
    fin = _tc3(a2[:NP], a2[NP:], d0, d1, b2.reshape(1, DIM_H),
               batchp, wlp, blp)
    return fin[:, :DIM_O]


# speedup vs baseline: 17.2615x; 17.2615x over previous
import functools

import jax
import jax.numpy as jnp
from jax import lax
from jax.experimental import pallas as pl
from jax.experimental.pallas import tpu as pltpu
from jax.experimental.pallas import tpu_sc as plsc

N_NODES = 10000
DIM_IN = 128
DIM_H = 64
DIM_O = 6
N_GRAPH = 64
N_EDGE = 320000

NC, NS, LANES = 2, 16, 16
NW = NC * NS
NP = 10240
ROWS_W = NP // NS
CH = 128
NCH_W = 82
EP = NW * NCH_W * CH
BLK = 640
GRID = NP // BLK

_mesh = plsc.VectorSubcoreMesh(
    core_axis_name="c", subcore_axis_name="s", num_cores=NC, num_subcores=NS)
_sc_params = pltpu.CompilerParams(use_tc_tiling_on_sc=False)


@functools.partial(
    pl.kernel,
    out_type=jax.ShapeDtypeStruct((NC * NP, LANES), jnp.float32),
    mesh=_mesh,
    scratch_types=[
        pltpu.VMEM((CH, LANES), jnp.float32),
        pltpu.VMEM((CH, LANES), jnp.float32),
        pltpu.VMEM((CH,), jnp.int32),
        pltpu.VMEM_SHARED((NP, LANES), jnp.float32),
    ],
    compiler_params=_sc_params,
)
def _deg_kernel(dst_hbm, out_hbm, zb, ones_v, didx, cnt_sp):
    c = lax.axis_index("c")
    s = lax.axis_index("s")
    w = c * NS + s

    def fill(i, _):
        zb[i, :] = jnp.zeros((LANES,), jnp.float32)
        ones_v[i, :] = jnp.ones((LANES,), jnp.float32)
        return 0

    lax.fori_loop(0, CH, fill, 0)
    for k in range(ROWS_W // CH):
        pltpu.sync_copy(zb, cnt_sp.at[pl.ds(s * ROWS_W + k * CH, CH)])
    plsc.subcore_barrier()

    def chunk(j, _):
        base = pl.multiple_of((w * NCH_W + j) * CH, CH)
        pltpu.sync_copy(dst_hbm.at[pl.ds(base, CH)], didx)
        pltpu.sync_copy(ones_v, cnt_sp.at[didx], add=True)
        return 0

    lax.fori_loop(0, NCH_W, chunk, 0)
    plsc.subcore_barrier()
    pltpu.sync_copy(cnt_sp.at[pl.ds(s * ROWS_W, ROWS_W)],
                    out_hbm.at[pl.ds(c * NP + s * ROWS_W, ROWS_W)])


@functools.partial(
    pl.kernel,
    out_type=jax.ShapeDtypeStruct((NC * NP, DIM_H), jnp.float32),
    mesh=_mesh,
    scratch_types=[
        pltpu.VMEM((CH, DIM_H), jnp.float32),
        pltpu.VMEM((2, CH, DIM_H), jnp.float32),
        pltpu.VMEM((2, CH), jnp.int32),
        pltpu.VMEM((2, CH), jnp.int32),
        pltpu.SemaphoreType.DMA,
        pltpu.VMEM_SHARED((NP, DIM_H), jnp.float32),
    ],
    compiler_params=_sc_params,
)
def _agg_kernel(g_hbm, src_hbm, dst_hbm, out_hbm, zb, rows, sidx, didx, sem,
                acc_sp):
    c = lax.axis_index("c")
    s = lax.axis_index("s")
    w = c * NS + s

    def fill(i, _):
        for k in range(DIM_H // LANES):
            zb[i, pl.ds(k * LANES, LANES)] = jnp.zeros((LANES,), jnp.float32)
        return 0

    lax.fori_loop(0, CH, fill, 0)
    for k in range(ROWS_W // CH):
        pltpu.sync_copy(zb, acc_sp.at[pl.ds(s * ROWS_W + k * CH, CH)])
    plsc.subcore_barrier()

    base0 = w * NCH_W * CH

    def chunk(j, _):
        base = pl.multiple_of(base0 + j * CH, CH)
        pltpu.sync_copy(src_hbm.at[pl.ds(base, CH)], sidx.at[0])
        pltpu.sync_copy(dst_hbm.at[pl.ds(base, CH)], didx.at[0])
        pltpu.async_copy(g_hbm.at[sidx.at[0]], rows.at[0], sem).wait()
        pltpu.sync_copy(rows.at[0], acc_sp.at[didx.at[0]], add=True)
        return 0

    lax.fori_loop(0, NCH_W, chunk, 0)
    plsc.subcore_barrier()
    pltpu.sync_copy(acc_sp.at[pl.ds(s * ROWS_W, ROWS_W)],
                    out_hbm.at[pl.ds(c * NP + s * ROWS_W, ROWS_W)])


def _dinv_block(d0_ref, d1_ref):
    i = pl.program_id(0)
    deg = d0_ref[:, 0:1] + d1_ref[:, 0:1]
    rows = i * BLK + lax.broadcasted_iota(jnp.int32, (BLK, 1), 0)
    ok = (rows < N_NODES) & (deg > 0.0)
    return jnp.where(ok, lax.rsqrt(jnp.maximum(deg, 1e-30)), 0.0)


def _tc1_body(x_ref, w1_ref, d0_ref, d1_ref, g_ref):
    dinv = _dinv_block(d0_ref, d1_ref)
    g = jnp.dot(x_ref[...], w1_ref[...], preferred_element_type=jnp.float32)
    g_ref[...] = g * dinv


def _tc1(xp, W1, d0, d1):
    return pl.pallas_call(
        _tc1_body,
        grid=(GRID,),
        in_specs=[
            pl.BlockSpec((BLK, DIM_IN), lambda i: (i, 0)),
            pl.BlockSpec((DIM_IN, DIM_H), lambda i: (0, 0)),
            pl.BlockSpec((BLK, LANES), lambda i: (i, 0)),
            pl.BlockSpec((BLK, LANES), lambda i: (i, 0)),
        ],
        out_specs=pl.BlockSpec((BLK, DIM_H), lambda i: (i, 0)),
        out_shape=jax.ShapeDtypeStruct((NP, DIM_H), jnp.float32),
    )(xp, W1, d0, d1)


def _tc2_body(p0_ref, p1_ref, d0_ref, d1_ref, b1_ref, w2_ref, g_ref):
    dinv = _dinv_block(d0_ref, d1_ref)
    a = (p0_ref[...] + p1_ref[...]) * dinv + b1_ref[...]
    h = jnp.maximum(a, 0.0)
    g_ref[...] = jnp.dot(h, w2_ref[...],
                         preferred_element_type=jnp.float32) * dinv


def _tc2(p0, p1, d0, d1, b1r, W2):
    return pl.pallas_call(
        _tc2_body,
        grid=(GRID,),
        in_specs=[
            pl.BlockSpec((BLK, DIM_H), lambda i: (i, 0)),
            pl.BlockSpec((BLK, DIM_H), lambda i: (i, 0)),
            pl.BlockSpec((BLK, LANES), lambda i: (i, 0)),
            pl.BlockSpec((BLK, LANES), lambda i: (i, 0)),
            pl.BlockSpec((1, DIM_H), lambda i: (0, 0)),
            pl.BlockSpec((DIM_H, DIM_H), lambda i: (0, 0)),
        ],
        out_specs=pl.BlockSpec((BLK, DIM_H), lambda i: (i, 0)),
        out_shape=jax.ShapeDtypeStruct((NP, DIM_H), jnp.float32),
    )(p0, p1, d0, d1, b1r, W2)


def _tc3_body(p0_ref, p1_ref, d0_ref, d1_ref, b2_ref, bt_ref, wl_ref, bl_ref,
              fin_ref, acc):
    i = pl.program_id(0)
    dinv = _dinv_block(d0_ref, d1_ref)
    h2 = jnp.maximum((p0_ref[...] + p1_ref[...]) * dinv + b2_ref[...], 0.0)
    bt = bt_ref[0]
    gids = lax.broadcasted_iota(jnp.int32, (N_GRAPH, BLK), 0)
    oh = (bt == gids).astype(jnp.float32)
    haug = jnp.concatenate([h2, jnp.ones((BLK, DIM_H), jnp.float32)], axis=1)
    part = jnp.dot(oh, haug, preferred_element_type=jnp.float32)

    @pl.when(i == 0)
    def _():
        acc[...] = part

    @pl.when(i > 0)
    def _():
        acc[...] += part

    @pl.when(i == GRID - 1)
    def _():
        sums = acc[:, :DIM_H]
        cnt = acc[:, DIM_H:DIM_H + 1]
        pooled = sums / jnp.maximum(cnt, 1.0)
        fin_ref[...] = jnp.dot(pooled, wl_ref[...],
                               preferred_element_type=jnp.float32) + bl_ref[...]


def _tc3(p0, p1, d0, d1, b2r, batchp, wlp, blp):
    return pl.pallas_call(
        _tc3_body,
        grid=(GRID,),
        in_specs=[
            pl.BlockSpec((BLK, DIM_H), lambda i: (i, 0)),
            pl.BlockSpec((BLK, DIM_H), lambda i: (i, 0)),
            pl.BlockSpec((BLK, LANES), lambda i: (i, 0)),
            pl.BlockSpec((BLK, LANES), lambda i: (i, 0)),
            pl.BlockSpec((1, DIM_H), lambda i: (0, 0)),
            pl.BlockSpec((1, 1, BLK), lambda i: (i, 0, 0)),
            pl.BlockSpec((DIM_H, 128), lambda i: (0, 0)),
            pl.BlockSpec((1, 128), lambda i: (0, 0)),
        ],
        out_specs=pl.BlockSpec((N_GRAPH, 128), lambda i: (0, 0)),
        out_shape=jax.ShapeDtypeStruct((N_GRAPH, 128), jnp.float32),
        scratch_shapes=[pltpu.VMEM((N_GRAPH, 128), jnp.float32)],
    )(p0, p1, d0, d1, b2r, batchp, wlp, blp)


def kernel(x, edge_index, batch, W1, b1, W2, b2, Wlin, blin):
    loop = jnp.arange(N_NODES, dtype=jnp.int32)
    npad = EP - (N_EDGE + N_NODES)
    pad_rows = N_NODES + (jnp.arange(npad, dtype=jnp.int32) % (NP - N_NODES))
    src = jnp.concatenate([edge_index[0], loop, pad_rows])
    dst = jnp.concatenate([edge_index[1], loop, pad_rows])

    deg2 = _deg_kernel(dst)
    d0, d1 = deg2[:NP], deg2[NP:]

    xp = jnp.pad(x, ((0, NP - N_NODES), (0, 0)))
    g1 = _tc1(xp, W1, d0, d1)
    a1 = _agg_kernel(g1, src, dst)
    g2 = _tc2(a1[:NP], a1[NP:], d0, d1, b1.reshape(1, DIM_H), W2)
    a2 = _agg_kernel(g2, src, dst)

    batchp = jnp.pad(batch, (0, NP - N_NODES),
                     constant_values=N_GRAPH).reshape(GRID, 1, BLK)
    wlp = jnp.pad(Wlin, ((0, 0), (0, 128 - DIM_O)))
    blp = jnp.pad(blin, (0, 128 - DIM_O)).reshape(1, 128)
    fin = _tc3(a2[:NP], a2[NP:], d0, d1, b2.reshape(1, DIM_H),
               batchp, wlp, blp)
    return fin[:, :DIM_O]


# speedup vs baseline: 37.6273x; 2.1798x over previous
import functools

import jax
import jax.numpy as jnp
from jax import lax
from jax.experimental import pallas as pl
from jax.experimental.pallas import tpu as pltpu
from jax.experimental.pallas import tpu_sc as plsc

N_NODES = 10000
DIM_IN = 128
DIM_H = 64
DIM_O = 6
N_GRAPH = 64
N_EDGE = 320000

NC, NS, LANES = 2, 16, 16
NW = NC * NS
NP = 10240
ROWS_W = NP // NS
CH = 128
NCH_W = 84
EP = NW * NCH_W * CH
NB = 4
BLK = 640
GRID = NP // BLK

_mesh = plsc.VectorSubcoreMesh(
    core_axis_name="c", subcore_axis_name="s", num_cores=NC, num_subcores=NS)
_sc_params = pltpu.CompilerParams(use_tc_tiling_on_sc=False)


@functools.partial(
    pl.kernel,
    out_type=jax.ShapeDtypeStruct((NC * NP, LANES), jnp.float32),
    mesh=_mesh,
    scratch_types=[
        pltpu.VMEM((CH, LANES), jnp.float32),
        pltpu.VMEM((CH, LANES), jnp.float32),
        pltpu.VMEM((NCH_W, CH), jnp.int32),
        pltpu.SemaphoreType.DMA,
        pltpu.VMEM_SHARED((NP, LANES), jnp.float32),
    ],
    compiler_params=_sc_params,
)
def _deg_kernel(dst_hbm, out_hbm, zb, ones_v, didx, ssem, cnt_sp):
    c = lax.axis_index("c")
    s = lax.axis_index("s")
    w = c * NS + s

    def fill(i, _):
        zb[i, :] = jnp.zeros((LANES,), jnp.float32)
        ones_v[i, :] = jnp.ones((LANES,), jnp.float32)
        return 0

    lax.fori_loop(0, CH, fill, 0)
    pltpu.sync_copy(dst_hbm.at[w], didx)
    for k in range(ROWS_W // CH):
        pltpu.sync_copy(zb, cnt_sp.at[pl.ds(s * ROWS_W + k * CH, CH)])
    plsc.subcore_barrier()

    def pair(i, _):
        for k in range(2):
            j = i * 2 + k
            pltpu.async_copy(ones_v, cnt_sp.at[didx.at[j]], ssem, add=True)

            @pl.when(j >= 4)
            def _():
                pltpu.make_async_copy(ones_v, cnt_sp.at[didx.at[j]],
                                      ssem).wait()
        return 0

    lax.fori_loop(0, NCH_W // 2, pair, 0)
    for _ in range(4):
        pltpu.make_async_copy(ones_v, cnt_sp.at[didx.at[0]], ssem).wait()
    plsc.subcore_barrier()
    pltpu.sync_copy(cnt_sp.at[pl.ds(s * ROWS_W, ROWS_W)],
                    out_hbm.at[pl.ds(c * NP + s * ROWS_W, ROWS_W)])


@functools.partial(
    pl.kernel,
    out_type=jax.ShapeDtypeStruct((NC * NP, DIM_H), jnp.float32),
    mesh=_mesh,
    scratch_types=[
        pltpu.VMEM((CH, DIM_H), jnp.float32),
        pltpu.VMEM((NB, CH, DIM_H), jnp.float32),
        pltpu.VMEM((NCH_W, CH), jnp.int32),
        pltpu.VMEM((NCH_W, CH), jnp.int32),
        pltpu.SemaphoreType.DMA,
        pltpu.VMEM_SHARED((NP, DIM_H), jnp.float32),
    ],
    compiler_params=_sc_params,
)
def _agg_kernel(g_hbm, src_hbm, dst_hbm, out_hbm, zb, rows, sidx, didx, gsem,
                acc_sp):
    c = lax.axis_index("c")
    s = lax.axis_index("s")
    w = c * NS + s

    def fill(i, _):
        for k in range(DIM_H // LANES):
            zb[i, pl.ds(k * LANES, LANES)] = jnp.zeros((LANES,), jnp.float32)
        return 0

    lax.fori_loop(0, CH, fill, 0)
    pltpu.sync_copy(src_hbm.at[w], sidx)
    pltpu.sync_copy(dst_hbm.at[w], didx)
    for k in range(ROWS_W // CH):
        pltpu.sync_copy(zb, acc_sp.at[pl.ds(s * ROWS_W + k * CH, CH)])
    plsc.subcore_barrier()

    for b in range(NB - 1):
        pltpu.async_copy(g_hbm.at[sidx.at[b]], rows.at[b], gsem)

    def quad(i, _):
        for k in range(NB):
            j = i * NB + k
            pltpu.make_async_copy(g_hbm.at[sidx.at[k]], rows.at[k],
                                  gsem).wait()
            jn = j + NB - 1

            @pl.when(jn < NCH_W)
            def _():
                bn = (k + NB - 1) % NB
                pltpu.async_copy(g_hbm.at[sidx.at[jn]], rows.at[bn], gsem)

            pltpu.sync_copy(rows.at[k], acc_sp.at[didx.at[j]], add=True)
        return 0

    lax.fori_loop(0, NCH_W // NB, quad, 0)
    plsc.subcore_barrier()
    pltpu.sync_copy(acc_sp.at[pl.ds(s * ROWS_W, ROWS_W)],
                    out_hbm.at[pl.ds(c * NP + s * ROWS_W, ROWS_W)])


def _dinv_block(d0_ref, d1_ref):
    i = pl.program_id(0)
    deg = d0_ref[:, 0:1] + d1_ref[:, 0:1]
    rows = i * BLK + lax.broadcasted_iota(jnp.int32, (BLK, 1), 0)
    ok = (rows < N_NODES) & (deg > 0.0)
    return jnp.where(ok, lax.rsqrt(jnp.maximum(deg, 1e-30)), 0.0)


def _tc1_body(x_ref, w1_ref, d0_ref, d1_ref, g_ref):
    dinv = _dinv_block(d0_ref, d1_ref)
    g = jnp.dot(x_ref[...], w1_ref[...], preferred_element_type=jnp.float32)
    g_ref[...] = g * dinv


def _tc1(xp, W1, d0, d1):
    return pl.pallas_call(
        _tc1_body,
        grid=(GRID,),
        in_specs=[
            pl.BlockSpec((BLK, DIM_IN), lambda i: (i, 0)),
            pl.BlockSpec((DIM_IN, DIM_H), lambda i: (0, 0)),
            pl.BlockSpec((BLK, LANES), lambda i: (i, 0)),
            pl.BlockSpec((BLK, LANES), lambda i: (i, 0)),
        ],
        out_specs=pl.BlockSpec((BLK, DIM_H), lambda i: (i, 0)),
        out_shape=jax.ShapeDtypeStruct((NP, DIM_H), jnp.float32),
    )(xp, W1, d0, d1)


def _tc2_body(p0_ref, p1_ref, d0_ref, d1_ref, b1_ref, w2_ref, g_ref):
    dinv = _dinv_block(d0_ref, d1_ref)
    a = (p0_ref[...] + p1_ref[...]) * dinv + b1_ref[...]
    h = jnp.maximum(a, 0.0)
    g_ref[...] = jnp.dot(h, w2_ref[...],
                         preferred_element_type=jnp.float32) * dinv


def _tc2(p0, p1, d0, d1, b1r, W2):
    return pl.pallas_call(
        _tc2_body,
        grid=(GRID,),
        in_specs=[
            pl.BlockSpec((BLK, DIM_H), lambda i: (i, 0)),
            pl.BlockSpec((BLK, DIM_H), lambda i: (i, 0)),
            pl.BlockSpec((BLK, LANES), lambda i: (i, 0)),
            pl.BlockSpec((BLK, LANES), lambda i: (i, 0)),
            pl.BlockSpec((1, DIM_H), lambda i: (0, 0)),
            pl.BlockSpec((DIM_H, DIM_H), lambda i: (0, 0)),
        ],
        out_specs=pl.BlockSpec((BLK, DIM_H), lambda i: (i, 0)),
        out_shape=jax.ShapeDtypeStruct((NP, DIM_H), jnp.float32),
    )(p0, p1, d0, d1, b1r, W2)


def _tc3_body(p0_ref, p1_ref, d0_ref, d1_ref, b2_ref, bt_ref, wl_ref, bl_ref,
              fin_ref, acc):
    i = pl.program_id(0)
    dinv = _dinv_block(d0_ref, d1_ref)
    h2 = jnp.maximum((p0_ref[...] + p1_ref[...]) * dinv + b2_ref[...], 0.0)
    bt = bt_ref[0]
    gids = lax.broadcasted_iota(jnp.int32, (N_GRAPH, BLK), 0)
    oh = (bt == gids).astype(jnp.float32)
    haug = jnp.concatenate([h2, jnp.ones((BLK, DIM_H), jnp.float32)], axis=1)
    part = jnp.dot(oh, haug, preferred_element_type=jnp.float32)

    @pl.when(i == 0)
    def _():
        acc[...] = part

    @pl.when(i > 0)
    def _():
        acc[...] += part

    @pl.when(i == GRID - 1)
    def _():
        sums = acc[:, :DIM_H]
        cnt = acc[:, DIM_H:DIM_H + 1]
        pooled = sums / jnp.maximum(cnt, 1.0)
        fin_ref[...] = jnp.dot(pooled, wl_ref[...],
                               preferred_element_type=jnp.float32) + bl_ref[...]


def _tc3(p0, p1, d0, d1, b2r, batchp, wlp, blp):
    return pl.pallas_call(
        _tc3_body,
        grid=(GRID,),
        in_specs=[
            pl.BlockSpec((BLK, DIM_H), lambda i: (i, 0)),
            pl.BlockSpec((BLK, DIM_H), lambda i: (i, 0)),
            pl.BlockSpec((BLK, LANES), lambda i: (i, 0)),
            pl.BlockSpec((BLK, LANES), lambda i: (i, 0)),
            pl.BlockSpec((1, DIM_H), lambda i: (0, 0)),
            pl.BlockSpec((1, 1, BLK), lambda i: (i, 0, 0)),
            pl.BlockSpec((DIM_H, 128), lambda i: (0, 0)),
            pl.BlockSpec((1, 128), lambda i: (0, 0)),
        ],
        out_specs=pl.BlockSpec((N_GRAPH, 128), lambda i: (0, 0)),
        out_shape=jax.ShapeDtypeStruct((N_GRAPH, 128), jnp.float32),
        scratch_shapes=[pltpu.VMEM((N_GRAPH, 128), jnp.float32)],
    )(p0, p1, d0, d1, b2r, batchp, wlp, blp)


def kernel(x, edge_index, batch, W1, b1, W2, b2, Wlin, blin):
    loop = jnp.arange(N_NODES, dtype=jnp.int32)
    npad = EP - (N_EDGE + N_NODES)
    pad_rows = N_NODES + (jnp.arange(npad, dtype=jnp.int32) % (NP - N_NODES))
    src = jnp.concatenate([edge_index[0], loop, pad_rows]
                          ).reshape(NCH_W, NW, CH).transpose(1, 0, 2)
    dst = jnp.concatenate([edge_index[1], loop, pad_rows]
                          ).reshape(NCH_W, NW, CH).transpose(1, 0, 2)

    deg2 = _deg_kernel(dst)
    d0, d1 = deg2[:NP], deg2[NP:]

    xp = jnp.pad(x, ((0, NP - N_NODES), (0, 0)))
    g1 = _tc1(xp, W1, d0, d1)
    a1 = _agg_kernel(g1, src, dst)
    g2 = _tc2(a1[:NP], a1[NP:], d0, d1, b1.reshape(1, DIM_H), W2)
    a2 = _agg_kernel(g2, src, dst)

    batchp = jnp.pad(batch, (0, NP - N_NODES),
                     constant_values=N_GRAPH).reshape(GRID, 1, BLK)
    wlp = jnp.pad(Wlin, ((0, 0), (0, 128 - DIM_O)))
    blp = jnp.pad(blin, (0, 128 - DIM_O)).reshape(1, 128)
    fin = _tc3(a2[:NP], a2[NP:], d0, d1, b2.reshape(1, DIM_H),
               batchp, wlp, blp)
    return fin[:, :DIM_O]
